# baseline (device time: 260895 ns/iter reference)
import jax
import jax.numpy as jnp
from jax import lax
from jax.experimental import pallas as pl
from jax.experimental.pallas import tpu as pltpu

N_DEV = 16


def _ring_allgather_weights(wq_shard, wo_shard):
    d_model, h_per = wq_shard.shape
    h_per_o, d_out = wo_shard.shape

    def body(wq_ref, wo_ref, wq_full, wo_full, q_send, q_recv, o_send, o_recv):
        my = lax.axis_index("i")
        left = lax.rem(my + N_DEV - 1, N_DEV)
        right = lax.rem(my + 1, N_DEV)

        barrier = pltpu.get_barrier_semaphore()
        pl.semaphore_signal(barrier, inc=1, device_id=(left,),
                            device_id_type=pl.DeviceIdType.MESH)
        pl.semaphore_signal(barrier, inc=1, device_id=(right,),
                            device_id_type=pl.DeviceIdType.MESH)
        pl.semaphore_wait(barrier, 2)

        wq_full[:, pl.ds(my * h_per, h_per)] = wq_ref[...]
        wo_full[pl.ds(my * h_per_o, h_per_o), :] = wo_ref[...]

        for h in range(N_DEV - 1):
            c = lax.rem(my - h + N_DEV, N_DEV)
            rq = pltpu.make_async_remote_copy(
                src_ref=wq_full.at[:, pl.ds(c * h_per, h_per)],
                dst_ref=wq_full.at[:, pl.ds(c * h_per, h_per)],
                send_sem=q_send.at[h],
                recv_sem=q_recv.at[h],
                device_id=(right,),
                device_id_type=pl.DeviceIdType.MESH,
            )
            ro = pltpu.make_async_remote_copy(
                src_ref=wo_full.at[pl.ds(c * h_per_o, h_per_o), :],
                dst_ref=wo_full.at[pl.ds(c * h_per_o, h_per_o), :],
                send_sem=o_send.at[h],
                recv_sem=o_recv.at[h],
                device_id=(right,),
                device_id_type=pl.DeviceIdType.MESH,
            )
            rq.start()
            ro.start()
            rq.wait()
            ro.wait()

    return pl.pallas_call(
        body,
        out_shape=[
            jax.ShapeDtypeStruct((d_model, N_DEV * h_per), wq_shard.dtype),
            jax.ShapeDtypeStruct((N_DEV * h_per_o, d_out), wo_shard.dtype),
        ],
        in_specs=[
            pl.BlockSpec(memory_space=pltpu.VMEM),
            pl.BlockSpec(memory_space=pltpu.VMEM),
        ],
        out_specs=[
            pl.BlockSpec(memory_space=pltpu.VMEM),
            pl.BlockSpec(memory_space=pltpu.VMEM),
        ],
        scratch_shapes=[
            pltpu.SemaphoreType.DMA((N_DEV - 1,)),
            pltpu.SemaphoreType.DMA((N_DEV - 1,)),
            pltpu.SemaphoreType.DMA((N_DEV - 1,)),
            pltpu.SemaphoreType.DMA((N_DEV - 1,)),
        ],
        compiler_params=pltpu.CompilerParams(collective_id=0),
    )(wq_shard, wo_shard)


def kernel(x, Wq, K_ext, V_ext, Wo):
    my = lax.axis_index("i")
    wq_full, wo_full = _ring_allgather_weights(Wq, Wo)

    B, Sq, Dm = x.shape
    _, Skv, Hq, Dh = K_ext.shape
    K_loc = lax.dynamic_slice_in_dim(K_ext, my * B, B, axis=0)
    V_loc = lax.dynamic_slice_in_dim(V_ext, my * B, B, axis=0)

    Q = (x.reshape(B * Sq, Dm) @ wq_full).reshape(B, Sq, Hq, Dh)
    qi = lax.broadcasted_iota(jnp.int32, (Sq, Skv), 0)
    ki = lax.broadcasted_iota(jnp.int32, (Sq, Skv), 1)
    mask = jnp.abs(qi - ki) <= 128

    scores = jnp.einsum("bihd,bjhd->bhij", Q, K_loc) * 0.125
    scores = jnp.where(mask[None, None], scores, -1e9)
    m = scores.max(axis=-1, keepdims=True)
    w = jnp.exp(scores - m)
    w = w / w.sum(axis=-1, keepdims=True)
    ctx = jnp.einsum("bhij,bjhd->bihd", w, V_loc).reshape(B, Sq, Hq * Dh)
    return ctx @ wo_full


# device time: 154073 ns/iter; 1.6933x vs baseline; 1.6933x over previous
import jax
import jax.numpy as jnp
from jax import lax
from jax.experimental import pallas as pl
from jax.experimental.pallas import tpu as pltpu

N_DEV = 16
R_HOPS = 8
L_HOPS = 7
B, SQ, SKV, DM = 2, 256, 256, 512
HQ, DH = 64, 64
HP = 256
HPH = HP // DH
WINDOW = 128
NEG = -1e9


def _body(x_ref, wq_ref, wo_ref, kt_ref, vt_ref, out_ref, wq_full, wo_full,
          qs_r, qr_r, os_r, or_r, qs_l, qr_l, os_l, or_l):
    my = lax.axis_index("i")
    left = lax.rem(my + N_DEV - 1, N_DEV)
    right = lax.rem(my + 1, N_DEV)

    barrier = pltpu.get_barrier_semaphore()
    pl.semaphore_signal(barrier, inc=1, device_id=(left,),
                        device_id_type=pl.DeviceIdType.MESH)
    pl.semaphore_signal(barrier, inc=1, device_id=(right,),
                        device_id_type=pl.DeviceIdType.MESH)
    pl.semaphore_wait(barrier, 2)

    def rdma(c, tgt, send_sem, recv_sem, is_wq):
        full = wq_full if is_wq else wo_full
        sl = (full.at[:, pl.ds(c * HP, HP)] if is_wq
              else full.at[pl.ds(c * HP, HP), :])
        return pltpu.make_async_remote_copy(
            src_ref=sl, dst_ref=sl,
            send_sem=send_sem, recv_sem=recv_sem,
            device_id=(tgt,), device_id_type=pl.DeviceIdType.MESH,
        )

    qi = lax.broadcasted_iota(jnp.int32, (SQ, SKV), 0)
    ki = lax.broadcasted_iota(jnp.int32, (SQ, SKV), 1)
    band = jnp.abs(qi - ki) <= WINDOW

    xflat = x_ref[...].reshape(B * SQ, DM)

    def compute_chunk(j):
        wq_j = wq_full[:, pl.ds(j * HP, HP)]
        wo_j = wo_full[pl.ds(j * HP, HP), :]
        q_all = jnp.dot(xflat, wq_j,
                        preferred_element_type=jnp.float32)
        ctx_rows = []
        for b in range(B):
            qb = q_all[b * SQ:(b + 1) * SQ, :]
            ctx_h = []
            for hh in range(HPH):
                jh = j * HPH + hh
                q = qb[:, hh * DH:(hh + 1) * DH]
                kt = kt_ref[b, pl.ds(jh, 1), :, :].reshape(DH, SKV)
                s = jnp.dot(q, kt,
                            preferred_element_type=jnp.float32) * 0.125
                s = jnp.where(band, s, NEG)
                m = jnp.max(s, axis=1, keepdims=True)
                e = jnp.exp(s - m)
                w = e / jnp.sum(e, axis=1, keepdims=True)
                v = vt_ref[b, pl.ds(jh, 1), :, :].reshape(SKV, DH)
                ctx_h.append(jnp.dot(w, v,
                                     preferred_element_type=jnp.float32))
            ctx_rows.append(jnp.concatenate(ctx_h, axis=1))
        ctx = jnp.concatenate(ctx_rows, axis=0)
        contrib = jnp.dot(ctx, wo_j,
                          preferred_element_type=jnp.float32)
        out_ref[...] = out_ref[...] + contrib.reshape(B, SQ, DM)

    wq_full[:, pl.ds(my * HP, HP)] = wq_ref[...]
    wo_full[pl.ds(my * HP, HP), :] = wo_ref[...]
    out_ref[...] = jnp.zeros((B, SQ, DM), jnp.float32)

    started = []

    def start_sends(h):
        if h < R_HOPS:
            c = lax.rem(my - h + N_DEV, N_DEV)
            for is_wq, ss, rs in ((True, qs_r, qr_r), (False, os_r, or_r)):
                r = rdma(c, right, ss.at[h], rs.at[h], is_wq)
                r.start()
                started.append(r)
        if h < L_HOPS:
            c = lax.rem(my + h, N_DEV)
            for is_wq, ss, rs in ((True, qs_l, qr_l), (False, os_l, or_l)):
                r = rdma(c, left, ss.at[h], rs.at[h], is_wq)
                r.start()
                started.append(r)

    start_sends(0)
    compute_chunk(my)

    for h in range(R_HOPS):
        c_r = lax.rem(my - h - 1 + N_DEV, N_DEV)
        rdma(c_r, left, qs_r.at[h], qr_r.at[h], True).wait_recv()
        rdma(c_r, left, os_r.at[h], or_r.at[h], False).wait_recv()
        if h < L_HOPS:
            c_l = lax.rem(my + h + 1, N_DEV)
            rdma(c_l, right, qs_l.at[h], qr_l.at[h], True).wait_recv()
            rdma(c_l, right, os_l.at[h], or_l.at[h], False).wait_recv()
        start_sends(h + 1)
        compute_chunk(c_r)
        if h < L_HOPS:
            compute_chunk(c_l)

    for r in started:
        r.wait_send()


def kernel(x, Wq, K_ext, V_ext, Wo):
    my = lax.axis_index("i")
    K_loc = lax.dynamic_slice_in_dim(K_ext, my * B, B, axis=0)
    V_loc = lax.dynamic_slice_in_dim(V_ext, my * B, B, axis=0)
    KT = K_loc.transpose(0, 2, 3, 1)
    VT = V_loc.transpose(0, 2, 1, 3)

    vmem = pl.BlockSpec(memory_space=pltpu.VMEM)
    return pl.pallas_call(
        _body,
        out_shape=jax.ShapeDtypeStruct((B, SQ, DM), jnp.float32),
        in_specs=[vmem] * 5,
        out_specs=vmem,
        scratch_shapes=[
            pltpu.VMEM((DM, N_DEV * HP), jnp.float32),
            pltpu.VMEM((N_DEV * HP, DM), jnp.float32),
            pltpu.SemaphoreType.DMA((R_HOPS,)),
            pltpu.SemaphoreType.DMA((R_HOPS,)),
            pltpu.SemaphoreType.DMA((R_HOPS,)),
            pltpu.SemaphoreType.DMA((R_HOPS,)),
            pltpu.SemaphoreType.DMA((L_HOPS,)),
            pltpu.SemaphoreType.DMA((L_HOPS,)),
            pltpu.SemaphoreType.DMA((L_HOPS,)),
            pltpu.SemaphoreType.DMA((L_HOPS,)),
        ],
        compiler_params=pltpu.CompilerParams(
            collective_id=0,
            vmem_limit_bytes=100 * 1024 * 1024,
        ),
    )(x, Wq, Wo, KT, VT)
